# baseline (speedup 1.0000x reference)
import jax
import jax.numpy as jnp
from jax import lax
from jax.experimental import pallas as pl

_INF = 1024.0
_BK = 8


def _fw_closed(D0, T):
    D = D0
    for k0 in range(0, T, _BK):
        R = lax.slice(D, (k0, 0), (k0 + _BK, T))
        for k in range(_BK):
            col = lax.slice(R, (0, k0 + k), (_BK, k0 + k + 1))
            row = lax.slice(R, (k, 0), (k + 1, T))
            R = jnp.minimum(R, col + row)
        U = None
        for k in range(_BK):
            colf = lax.slice(D, (0, k0 + k), (T, k0 + k + 1))
            rowf = lax.slice(R, (k, 0), (k + 1, T))
            term = colf + rowf
            U = term if U is None else jnp.minimum(U, term)
        D = jnp.minimum(D, U)
    return D


def _batch_distances(x, tb, mono_col, N, T):
    lane = lax.broadcasted_iota(jnp.int32, (N, T), 1)
    m = jnp.max(x, axis=1, keepdims=True)
    idx = jnp.min(jnp.where(x == m, lane, T), axis=1, keepdims=True)
    P = (lane == idx).astype(jnp.float32)

    gly_col = (mono_col != -1)
    Pg = P * gly_col.astype(jnp.float32)
    occ = jnp.max(Pg, axis=0, keepdims=True)

    D0 = jnp.where((tb > 0.0) & (occ > 0.5), 1.0, _INF)
    D = _fw_closed(D0, T)
    return P, D, gly_col


def _expand(P, D, gly_col, gly_row, N):
    R = lax.dot_general(P, D, (((1,), (0,)), ((), ())),
                        preferred_element_type=jnp.float32)
    O = lax.dot_general(R, P, (((1,), (1,)), ((), ())),
                        preferred_element_type=jnp.float32)
    li2 = lax.broadcasted_iota(jnp.int32, (N, N), 1)
    si2 = lax.broadcasted_iota(jnp.int32, (N, N), 0)
    vals = O.astype(jnp.int32)
    vals = jnp.where(O > 500.0, -1, vals)
    vals = jnp.where(gly_col & gly_row, vals, -1)
    vals = jnp.where(li2 == si2, 0, vals)
    return vals


def _fw_body(mono_col_ref, mono_row_ref, tb_ref, a2t_ref, out_ref):
    B = a2t_ref.shape[0]
    N = a2t_ref.shape[1]
    T = a2t_ref.shape[2]
    for b in range(B):
        P, D, gly_col = _batch_distances(
            a2t_ref[b], tb_ref[b], mono_col_ref[b], N, T)
        out_ref[b] = _expand(P, D, gly_col, mono_row_ref[b] != -1, N)


def kernel(atom_pad_mask, atom_mono_idx, token_bonds, atom_to_token):
    B, N = atom_pad_mask.shape
    T = token_bonds.shape[1]
    tb = jnp.squeeze(token_bonds, -1)
    mono_col = atom_mono_idx.reshape(B, N, 1)
    mono_row = atom_mono_idx.reshape(B, 1, N)
    out = pl.pallas_call(
        _fw_body,
        out_shape=jax.ShapeDtypeStruct((B, N, N), jnp.int32),
    )(mono_col, mono_row, tb, atom_to_token)
    return out


# speedup vs baseline: 2.5814x; 2.5814x over previous
import jax
import jax.numpy as jnp
from jax import lax
from jax.experimental import pallas as pl

_INF = 1024.0
_BK = 8


def _bfs_closed(adj, occ, T):
    Af = jnp.where(adj, 1.0, 0.0).astype(jnp.bfloat16)
    Bf = jnp.where(adj & (occ > 0.5), 1.0, 0.0).astype(jnp.bfloat16)
    dist0 = jnp.where(adj, 1.0, _INF)

    def cond(c):
        d, changed, _, _ = c
        return (d < T) & (changed > 0.5)

    def body(c):
        d, _, F, dist = c
        G = lax.dot_general(Bf, F, (((1,), (0,)), ((), ())),
                            preferred_element_type=jnp.float32)
        Fn = jnp.minimum(jnp.maximum(F, G.astype(jnp.bfloat16)),
                         jnp.bfloat16(1.0))
        isinf = jnp.floor(dist * (1.0 / _INF))
        newlyf = isinf * Fn.astype(jnp.float32)
        changed = jnp.max(newlyf)
        dist = dist + newlyf * ((d + 1).astype(jnp.float32) - _INF)
        return d + 1, changed, Fn, dist

    _, _, _, dist = lax.while_loop(
        cond, body, (jnp.int32(1), jnp.float32(1.0), Af, dist0))
    return dist


def _batch_distances(x, tb, mono_col, N, T):
    lane = lax.broadcasted_iota(jnp.int32, (N, T), 1)
    m = jnp.max(x, axis=1, keepdims=True)
    idx = jnp.min(jnp.where(x == m, lane, T), axis=1, keepdims=True)
    P = (lane == idx).astype(jnp.float32)

    gly_col = (mono_col != -1)
    Pg = P * gly_col.astype(jnp.float32)
    occ = jnp.max(Pg, axis=0, keepdims=True)

    D = _bfs_closed(tb > 0.0, occ, T)
    return P, D, gly_col


def _expand(P, D, gly_col, gly_row, N):
    R = lax.dot_general(P, D, (((1,), (0,)), ((), ())),
                        preferred_element_type=jnp.float32)
    O = lax.dot_general(R, P, (((1,), (1,)), ((), ())),
                        preferred_element_type=jnp.float32)
    li2 = lax.broadcasted_iota(jnp.int32, (N, N), 1)
    si2 = lax.broadcasted_iota(jnp.int32, (N, N), 0)
    vals = O.astype(jnp.int32)
    vals = jnp.where(O > 500.0, -1, vals)
    vals = jnp.where(gly_col & gly_row, vals, -1)
    vals = jnp.where(li2 == si2, 0, vals)
    return vals


def _fw_body(mono_col_ref, mono_row_ref, tb_ref, a2t_ref, out_ref):
    B = a2t_ref.shape[0]
    N = a2t_ref.shape[1]
    T = a2t_ref.shape[2]
    for b in range(B):
        P, D, gly_col = _batch_distances(
            a2t_ref[b], tb_ref[b], mono_col_ref[b], N, T)
        out_ref[b] = _expand(P, D, gly_col, mono_row_ref[b] != -1, N)


def kernel(atom_pad_mask, atom_mono_idx, token_bonds, atom_to_token):
    B, N = atom_pad_mask.shape
    T = token_bonds.shape[1]
    tb = jnp.squeeze(token_bonds, -1)
    mono_col = atom_mono_idx.reshape(B, N, 1)
    mono_row = atom_mono_idx.reshape(B, 1, N)
    out = pl.pallas_call(
        _fw_body,
        out_shape=jax.ShapeDtypeStruct((B, N, N), jnp.int32),
    )(mono_col, mono_row, tb, atom_to_token)
    return out


# speedup vs baseline: 2.6366x; 1.0214x over previous
import jax
import jax.numpy as jnp
from jax import lax
from jax.experimental import pallas as pl

_INF = 1024.0
_BK = 8


def _bfs_closed(adj, occ, T):
    Af = jnp.where(adj, 1.0, 0.0).astype(jnp.bfloat16)
    Bf = jnp.where(adj & (occ > 0.5), 1.0, 0.0).astype(jnp.bfloat16)
    dist0 = jnp.where(adj, 1.0, _INF)

    def cond(c):
        d, changed, _, _ = c
        return (d < T) & (changed > 0.5)

    def body(c):
        d, _, F, dist = c
        G = lax.dot_general(Bf, F, (((1,), (0,)), ((), ())),
                            preferred_element_type=jnp.float32)
        Fn = jnp.minimum(jnp.maximum(F, G.astype(jnp.bfloat16)),
                         jnp.bfloat16(1.0))
        isinf = jnp.floor(dist * (1.0 / _INF))
        newlyf = isinf * Fn.astype(jnp.float32)
        changed = jnp.max(newlyf)
        dist = dist + newlyf * ((d + 1).astype(jnp.float32) - _INF)
        return d + 1, changed, Fn, dist

    _, _, _, dist = lax.while_loop(
        cond, body, (jnp.int32(1), jnp.float32(1.0), Af, dist0))
    return dist


def _batch_distances(x, tb, mono_col, N, T):
    lane = lax.broadcasted_iota(jnp.int32, (N, T), 1)
    m = jnp.max(x, axis=1, keepdims=True)
    idx = jnp.min(jnp.where(x == m, lane, T), axis=1, keepdims=True)
    P = (lane == idx).astype(jnp.bfloat16)

    gly_col = (mono_col != -1)
    Pg = P * gly_col.astype(jnp.bfloat16)
    occ = jnp.max(Pg.astype(jnp.float32), axis=0, keepdims=True)

    D = _bfs_closed(tb > 0.0, occ, T)
    return P, D, gly_col


def _expand(P, D, gly_col, gly_row, N):
    R = lax.dot_general(P, D.astype(jnp.bfloat16), (((1,), (0,)), ((), ())),
                        preferred_element_type=jnp.float32)
    O = lax.dot_general(R.astype(jnp.bfloat16), P, (((1,), (1,)), ((), ())),
                        preferred_element_type=jnp.float32)
    li2 = lax.broadcasted_iota(jnp.int32, (N, N), 1)
    si2 = lax.broadcasted_iota(jnp.int32, (N, N), 0)
    vals = O.astype(jnp.int32)
    vals = jnp.where(O > 500.0, -1, vals)
    vals = jnp.where(gly_col & gly_row, vals, -1)
    vals = jnp.where(li2 == si2, 0, vals)
    return vals


def _fw_body(mono_col_ref, mono_row_ref, tb_ref, a2t_ref, out_ref):
    N = a2t_ref.shape[1]
    T = a2t_ref.shape[2]
    P, D, gly_col = _batch_distances(
        a2t_ref[0], tb_ref[0], mono_col_ref[0], N, T)
    out_ref[0] = _expand(P, D, gly_col, mono_row_ref[0] != -1, N)


def kernel(atom_pad_mask, atom_mono_idx, token_bonds, atom_to_token):
    B, N = atom_pad_mask.shape
    T = token_bonds.shape[1]
    tb = jnp.squeeze(token_bonds, -1)
    mono_col = atom_mono_idx.reshape(B, N, 1)
    mono_row = atom_mono_idx.reshape(B, 1, N)
    out = pl.pallas_call(
        _fw_body,
        grid=(B,),
        in_specs=[
            pl.BlockSpec((1, N, 1), lambda b: (b, 0, 0)),
            pl.BlockSpec((1, 1, N), lambda b: (b, 0, 0)),
            pl.BlockSpec((1, T, T), lambda b: (b, 0, 0)),
            pl.BlockSpec((1, N, T), lambda b: (b, 0, 0)),
        ],
        out_specs=pl.BlockSpec((1, N, N), lambda b: (b, 0, 0)),
        out_shape=jax.ShapeDtypeStruct((B, N, N), jnp.int32),
    )(mono_col, mono_row, tb, atom_to_token)
    return out
